# baseline (device time: 87731 ns/iter reference)
import jax
import jax.numpy as jnp
from jax import lax
from jax.experimental import pallas as pl
from jax.experimental.pallas import tpu as pltpu

X_SIZE = 2
NC = 4
C2 = 320


def kernel(x, assign, W1, W2):
    t_per, d = x.shape
    e_loc, _, f = W1.shape
    tc = t_per // NC
    CE = 2 * C2
    S = e_loc * CE

    a_col = assign.reshape(t_per, 1)
    a_row = assign.reshape(1, t_per)

    def body(x_ref, ac_ref, ar_ref, w1_ref, w2_ref, out_ref,
             xsend, xrecv, arc, arr, xg, ptl, ptlT, ptp, ptpT,
             res_send, res_recv, sems, rs_sems, rr_sems):
        my_x = lax.axis_index("x")
        my_y = lax.axis_index("y")
        my_z = lax.axis_index("z")
        peer = (1 - my_x, my_y, my_z)

        barrier = pltpu.get_barrier_semaphore()
        pl.semaphore_signal(barrier, inc=1, device_id=peer,
                            device_id_type=pl.DeviceIdType.MESH)
        pl.semaphore_wait(barrier, 1)

        xsend[...] = x_ref[...].astype(jnp.bfloat16)
        rdma_x = pltpu.make_async_remote_copy(
            src_ref=xsend, dst_ref=xrecv,
            send_sem=sems.at[0], recv_sem=sems.at[1],
            device_id=peer, device_id_type=pl.DeviceIdType.MESH)
        rdma_x.start()
        rdma_ac = pltpu.make_async_remote_copy(
            src_ref=ac_ref, dst_ref=arc,
            send_sem=sems.at[2], recv_sem=sems.at[3],
            device_id=peer, device_id_type=pl.DeviceIdType.MESH)
        rdma_ac.start()
        rdma_ar = pltpu.make_async_remote_copy(
            src_ref=ar_ref, dst_ref=arr,
            send_sem=sems.at[4], recv_sem=sems.at[5],
            device_id=peer, device_id_type=pl.DeviceIdType.MESH)
        rdma_ar.start()

        res_rdmas = [
            pltpu.make_async_remote_copy(
                src_ref=res_send.at[c], dst_ref=res_recv.at[c],
                send_sem=rs_sems.at[c], recv_sem=rr_sems.at[c],
                device_id=peer, device_id_type=pl.DeviceIdType.MESH)
            for c in range(NC)
        ]

        def build_pt(acol, arow, half_off, pt_ref, ptT_ref):
            eq = acol == arow
            ri = lax.broadcasted_iota(jnp.int32, (t_per, t_per), 0)
            ci = lax.broadcasted_iota(jnp.int32, (t_per, t_per), 1)
            prior = jnp.logical_and(eq, ci < ri).astype(jnp.int32)
            pos_c = jnp.sum(prior, axis=1, keepdims=True)
            pos_r = jnp.sum(
                jnp.logical_and(eq, ri < ci).astype(jnp.int32),
                axis=0, keepdims=True)
            loc_c = acol - my_x * e_loc
            loc_r = arow - my_x * e_loc
            mine_c = jnp.logical_and(loc_c >= 0, loc_c < e_loc)
            mine_r = jnp.logical_and(loc_r >= 0, loc_r < e_loc)
            slot_c = jnp.where(mine_c, loc_c * CE + half_off + pos_c, -1)
            slot_r = jnp.where(mine_r, loc_r * CE + half_off + pos_r, -1)
            sc = lax.broadcasted_iota(jnp.int32, (t_per, S), 1)
            sr = lax.broadcasted_iota(jnp.int32, (S, t_per), 0)
            pt_ref[...] = (slot_c == sc).astype(jnp.bfloat16)
            ptT_ref[...] = (sr == slot_r).astype(jnp.bfloat16)

        def mm(a, b):
            return lax.dot_general(a, b, (((1,), (0,)), ((), ())),
                                   preferred_element_type=jnp.float32)

        def expert_block(e, row_off):
            rows = pl.ds(e * CE + row_off, C2)
            h = jnp.maximum(mm(xg[rows, :], w1_ref[e].astype(jnp.bfloat16)),
                            0.0).astype(jnp.bfloat16)
            o = mm(h, w2_ref[e].astype(jnp.bfloat16))
            xg[rows, :] = o.astype(jnp.bfloat16)

        build_pt(ac_ref[...], ar_ref[...], 0, ptl, ptlT)
        xg[...] = mm(ptlT[...], xsend[...]).astype(jnp.bfloat16)
        for e in range(e_loc):
            expert_block(e, 0)

        rdma_x.wait()
        rdma_ac.wait()
        rdma_ar.wait()
        build_pt(arc[...], arr[...], C2, ptp, ptpT)
        xg[...] += mm(ptpT[...], xrecv[...]).astype(jnp.bfloat16)
        for e in range(e_loc):
            expert_block(e, C2)

        for c in range(NC):
            rows = pl.ds(c * tc, tc)
            res_send[c] = mm(ptp[rows, :], xg[...]).astype(jnp.bfloat16)
            res_rdmas[c].start()
        out_ref[...] = mm(ptl[...], xg[...])
        for c in range(NC):
            res_rdmas[c].wait_recv()
            rows = pl.ds(c * tc, tc)
            out_ref[rows, :] += res_recv[c].astype(jnp.float32)
        for c in range(NC):
            res_rdmas[c].wait_send()

    return pl.pallas_call(
        body,
        out_shape=jax.ShapeDtypeStruct((t_per, d), jnp.float32),
        in_specs=[pl.BlockSpec(memory_space=pltpu.VMEM)] * 5,
        out_specs=pl.BlockSpec(memory_space=pltpu.VMEM),
        scratch_shapes=[
            pltpu.VMEM((t_per, d), jnp.bfloat16),
            pltpu.VMEM((t_per, d), jnp.bfloat16),
            pltpu.VMEM((t_per, 1), jnp.int32),
            pltpu.VMEM((1, t_per), jnp.int32),
            pltpu.VMEM((S, d), jnp.bfloat16),
            pltpu.VMEM((t_per, S), jnp.bfloat16),
            pltpu.VMEM((S, t_per), jnp.bfloat16),
            pltpu.VMEM((t_per, S), jnp.bfloat16),
            pltpu.VMEM((S, t_per), jnp.bfloat16),
            pltpu.VMEM((NC, tc, d), jnp.bfloat16),
            pltpu.VMEM((NC, tc, d), jnp.bfloat16),
            pltpu.SemaphoreType.DMA((6,)),
            pltpu.SemaphoreType.DMA((NC,)),
            pltpu.SemaphoreType.DMA((NC,)),
        ],
        compiler_params=pltpu.CompilerParams(
            collective_id=0,
            vmem_limit_bytes=100 * 1024 * 1024,
        ),
    )(x, a_col, a_row, W1, W2)
